# Initial kernel scaffold; baseline (speedup 1.0000x reference)
import jax
import jax.numpy as jnp
from jax.experimental import pallas as pl


def kernel(states_x, states_edge_index_0, states_edge_index_1, states_edge_index_2, states_batch, goal_x, goal_edge_index_0, goal_edge_index_1, goal_edge_index_2, goal_batch, params):
    raise NotImplementedError("write your pallas kernel here")


# speedup vs baseline: 1.0869x; 1.0869x over previous
import jax
import jax.numpy as jnp
import numpy as np
from jax.experimental import pallas as pl
from jax.experimental.pallas import tpu as pltpu

N_REL = 3
IN_FEAT = 128
EMB = 64
N_LAYERS = 4
HID = 128
N_LSTM = 2
OUT_DIM = 256
B = 8
L = 10


def _mm_kernel(x_ref, w_ref, b_ref, o_ref):
    o_ref[...] = jnp.dot(x_ref[...], w_ref[...],
                         preferred_element_type=jnp.float32) + b_ref[...]


def _mm(x, w, b, block_rows=1000):
    n = x.shape[0]
    fout = w.shape[1]
    grid = (n // block_rows,)
    return pl.pallas_call(
        _mm_kernel,
        grid=grid,
        in_specs=[
            pl.BlockSpec((block_rows, x.shape[1]), lambda i: (i, 0)),
            pl.BlockSpec((x.shape[1], fout), lambda i: (0, 0)),
            pl.BlockSpec((1, fout), lambda i: (0, 0)),
        ],
        out_specs=pl.BlockSpec((block_rows, fout), lambda i: (i, 0)),
        out_shape=jax.ShapeDtypeStruct((n, fout), jnp.float32),
    )(x, w, b.reshape(1, -1))


def _encode(x, eidx, batch, num_graphs, params):
    x = _mm(x, params['emb_W'], params['emb_b'])
    n = x.shape[0]
    for l in range(N_LAYERS):
        x_out = x @ params['root_W'][l] + params['root_b'][l]
        for r in range(N_REL):
            xr = x @ params['conv_W'][l][r]
            src = eidx[r][0]
            dst = eidx[r][1]
            agg = jax.ops.segment_max(xr[src], dst, num_segments=n)
            agg = jnp.where(jnp.isneginf(agg), 0.0, agg)
            x_out = x_out + agg
        x = jax.nn.relu(x_out)
    return jax.ops.segment_sum(x, batch, num_segments=num_graphs)


def _lstm_layer(x, Wih, Whh, bih, bhh):
    h0 = jnp.zeros((x.shape[0], HID), x.dtype)
    c0 = jnp.zeros((x.shape[0], HID), x.dtype)

    def step(carry, xt):
        h, c = carry
        g = xt @ Wih.T + h @ Whh.T + bih + bhh
        i, f, gg, o = jnp.split(g, 4, axis=-1)
        i = jax.nn.sigmoid(i)
        f = jax.nn.sigmoid(f)
        gg = jnp.tanh(gg)
        o = jax.nn.sigmoid(o)
        c = f * c + i * gg
        h = o * jnp.tanh(c)
        return (h, c), h

    _, ys = jax.lax.scan(step, (h0, c0), jnp.swapaxes(x, 0, 1))
    return jnp.swapaxes(ys, 0, 1)


def kernel(states_x, states_edge_index_0, states_edge_index_1,
           states_edge_index_2, states_batch, goal_x, goal_edge_index_0,
           goal_edge_index_1, goal_edge_index_2, goal_batch, params):
    s_eidx = [states_edge_index_0, states_edge_index_1, states_edge_index_2]
    g_eidx = [goal_edge_index_0, goal_edge_index_1, goal_edge_index_2]
    state_emb = _encode(states_x, s_eidx, states_batch, B * L, params)
    goal_emb = _encode(goal_x, g_eidx, goal_batch, B, params)
    lengths = [L] * B
    padded = state_emb.reshape(B, L, EMB)
    input_emb = padded[:, :-1, :]
    target_emb = padded[:, 1:, :]
    mask = np.arange(L - 1)[None, :] < (np.array(lengths)[:, None] - 1)
    idx = np.nonzero(mask.reshape(-1))[0]
    goal_exp = jnp.repeat(goal_emb[:, None, :], L - 1, axis=1)
    out = jnp.concatenate([input_emb, goal_exp], axis=2)
    for l in range(N_LSTM):
        out = _lstm_layer(out, params['lstm_Wih'][l], params['lstm_Whh'][l],
                          params['lstm_bih'][l], params['lstm_bhh'][l])
    sel = out.reshape(B * (L - 1), HID)[idx]
    pred = sel @ params['head_W'] + params['head_b']
    h = jax.nn.relu(pred @ params['dec_W1'] + params['dec_b1'])
    logits = h @ params['dec_W2'] + params['dec_b2']
    target_flat = target_emb.reshape(B * (L - 1), EMB)[idx]
    return pred, target_flat, logits
